# baseline (speedup 1.0000x reference)
import functools

import jax
import jax.numpy as jnp
from jax import lax
from jax.experimental import pallas as pl
from jax.experimental.pallas import tpu as pltpu
from jax.experimental.pallas import tpu_sc as plsc

_ROWS, _COLS = 16384, 256
_NC, _NS = 2, 16
_NW = _NC * _NS
_ROWS_PER_W = _ROWS // _NW
_CHUNK = 256


def _sc_copy(x_hbm, o_hbm, buf0, buf1, si0, si1, so0, so1):
    wid = lax.axis_index("s") * _NC + lax.axis_index("c")
    base = wid * _ROWS_PER_W
    in0 = pltpu.make_async_copy(x_hbm.at[pl.ds(base, _CHUNK)], buf0, si0)
    in1 = pltpu.make_async_copy(
        x_hbm.at[pl.ds(base + _CHUNK, _CHUNK)], buf1, si1
    )
    in0.start()
    in1.start()
    in0.wait()
    out0 = pltpu.make_async_copy(buf0, o_hbm.at[pl.ds(base, _CHUNK)], so0)
    out0.start()
    in1.wait()
    out1 = pltpu.make_async_copy(
        buf1, o_hbm.at[pl.ds(base + _CHUNK, _CHUNK)], so1
    )
    out1.start()
    out0.wait()
    out1.wait()


def kernel(x):
    mesh = plsc.VectorSubcoreMesh(core_axis_name="c", subcore_axis_name="s")
    f = functools.partial(
        pl.kernel,
        out_type=jax.ShapeDtypeStruct((_ROWS, _COLS), jnp.float32),
        mesh=mesh,
        scratch_types=[
            pltpu.VMEM((_CHUNK, _COLS), jnp.float32),
            pltpu.VMEM((_CHUNK, _COLS), jnp.float32),
            pltpu.SemaphoreType.DMA,
            pltpu.SemaphoreType.DMA,
            pltpu.SemaphoreType.DMA,
            pltpu.SemaphoreType.DMA,
        ],
    )(_sc_copy)
    return f(x)


# speedup vs baseline: 2.9001x; 2.9001x over previous
import jax
import jax.numpy as jnp
from jax.experimental import pallas as pl
from jax.experimental.pallas import tpu as pltpu


_CHUNK_ROWS = (512, 1024, 2048, 4096, 4096, 2048, 1024, 1024, 512)
_N_CHUNKS = len(_CHUNK_ROWS)
_CHUNK_OFF = tuple(sum(_CHUNK_ROWS[:i]) for i in range(_N_CHUNKS))


def _copy_kernel(x_ref, o_ref, buf, sem_in, sem_out):
    ins = [
        pltpu.make_async_copy(
            x_ref.at[pl.ds(_CHUNK_OFF[c], _CHUNK_ROWS[c])],
            buf.at[pl.ds(_CHUNK_OFF[c], _CHUNK_ROWS[c])],
            sem_in.at[c],
        )
        for c in range(_N_CHUNKS)
    ]
    outs = [
        pltpu.make_async_copy(
            buf.at[pl.ds(_CHUNK_OFF[c], _CHUNK_ROWS[c])],
            o_ref.at[pl.ds(_CHUNK_OFF[c], _CHUNK_ROWS[c])],
            sem_out.at[c],
        )
        for c in range(_N_CHUNKS)
    ]
    for cp in ins:
        cp.start()
    for c in range(_N_CHUNKS):
        ins[c].wait()
        outs[c].start()
    for cp in outs:
        cp.wait()


def kernel(x):
    return pl.pallas_call(
        _copy_kernel,
        out_shape=jax.ShapeDtypeStruct(x.shape, x.dtype),
        in_specs=[pl.BlockSpec(memory_space=pl.ANY)],
        out_specs=pl.BlockSpec(memory_space=pl.ANY),
        scratch_shapes=[
            pltpu.VMEM(x.shape, x.dtype),
            pltpu.SemaphoreType.DMA((_N_CHUNKS,)),
            pltpu.SemaphoreType.DMA((_N_CHUNKS,)),
        ],
    )(x)
